# baseline (speedup 1.0000x reference)
import jax
import jax.numpy as jnp
from jax.experimental import pallas as pl
from jax.experimental.pallas import tpu as pltpu


def _moe_block(x_ref, u_ref, wg_ref, wexp_ref, bexp_ref, o_ref,
               yacc_ref, gacc_ref):
    Cs, Ns, P = x_ref.shape[1], x_ref.shape[2], x_ref.shape[3]
    E = u_ref.shape[3]
    T = Ns * P
    k = pl.program_id(2)
    nk = pl.num_programs(2)

    xb16 = x_ref[0].astype(jnp.bfloat16).reshape(Cs, T)
    g_part = jnp.dot(wg_ref[...].astype(jnp.bfloat16), xb16,
                     preferred_element_type=jnp.float32)
    y_part = jnp.dot(wexp_ref[...].astype(jnp.bfloat16), xb16,
                     preferred_element_type=jnp.float32)

    @pl.when(k == 0)
    def _():
        gacc_ref[...] = g_part
        yacc_ref[...] = y_part

    @pl.when(k > 0)
    def _():
        gacc_ref[...] += g_part
        yacc_ref[...] += y_part

    @pl.when(k == nk - 1)
    def _():
        g = gacc_ref[...]
        el = g[:E]
        nl = g[E:]
        nl_max = jnp.max(nl, axis=0, keepdims=True)
        nl_exp = jnp.exp(nl - nl_max)
        ut = u_ref[0].reshape(T, E).T
        noise = ut * (nl_exp / jnp.sum(nl_exp, axis=0, keepdims=True))
        logits = el + noise

        lmin = jnp.min(logits, axis=0, keepdims=True)
        eidx = jax.lax.broadcasted_iota(jnp.int32, logits.shape, 0)
        drop = jnp.max(jnp.where(logits == lmin, eidx, -1), axis=0,
                       keepdims=True)
        keep = eidx != drop
        lmax = jnp.max(logits, axis=0, keepdims=True)
        ex = jnp.where(keep, jnp.exp(logits - lmax), 0.0)
        gates = ex / jnp.sum(ex, axis=0, keepdims=True)
        s = jnp.sum(gates, axis=0, keepdims=True)

        yo = (yacc_ref[...] + bexp_ref[...]) * s
        o_ref[0] = yo.reshape(yo.shape[0], Ns, P)


def kernel(x, We, be, Wn, bn, Wexp, bexp, noise_uniform):
    B, C, N, P = x.shape
    E = We.shape[0]
    O = Wexp.shape[0]
    Ns = 32
    Cs = 256
    G = C // Cs

    u4 = noise_uniform.reshape(B, N, P, E)
    bexp2 = bexp.reshape(O, 1)
    Wg = jnp.concatenate([We, Wn], axis=0)

    out = pl.pallas_call(
        _moe_block,
        grid=(B, N // Ns, G),
        in_specs=[
            pl.BlockSpec((1, Cs, Ns, P), lambda b, j, k: (b, k, j, 0)),
            pl.BlockSpec((1, Ns, P, E), lambda b, j, k: (b, j, 0, 0)),
            pl.BlockSpec((2 * E, Cs), lambda b, j, k: (0, k)),
            pl.BlockSpec((O, Cs), lambda b, j, k: (0, k)),
            pl.BlockSpec((O, 1), lambda b, j, k: (0, 0)),
        ],
        out_specs=pl.BlockSpec((1, O, Ns, P), lambda b, j, k: (b, 0, j, 0)),
        out_shape=jax.ShapeDtypeStruct((B, O, N, P), x.dtype),
        scratch_shapes=[
            pltpu.VMEM((O, Ns * P), jnp.float32),
            pltpu.VMEM((2 * E, Ns * P), jnp.float32),
        ],
    )(x, u4, Wg, Wexp, bexp2)
    return out


# speedup vs baseline: 1.2309x; 1.2309x over previous
import jax
import jax.numpy as jnp
from jax.experimental import pallas as pl


def _moe_block(x_ref, u_ref, wg_ref, wexp_ref, bexp_ref, o_ref):
    xb16 = x_ref[0].astype(jnp.bfloat16)
    E = u_ref.shape[2]
    ut = u_ref[0].T

    g = jnp.dot(wg_ref[...].astype(jnp.bfloat16), xb16,
                preferred_element_type=jnp.float32)
    el = g[:E]
    nl = g[E:]
    nl_max = jnp.max(nl, axis=0, keepdims=True)
    nl_exp = jnp.exp(nl - nl_max)
    noise = ut * (nl_exp / jnp.sum(nl_exp, axis=0, keepdims=True))
    logits = el + noise

    lmin = jnp.min(logits, axis=0, keepdims=True)
    eidx = jax.lax.broadcasted_iota(jnp.int32, logits.shape, 0)
    drop = jnp.max(jnp.where(logits == lmin, eidx, -1), axis=0, keepdims=True)
    keep = eidx != drop
    lmax = jnp.max(logits, axis=0, keepdims=True)
    ex = jnp.where(keep, jnp.exp(logits - lmax), 0.0)
    gates = ex / jnp.sum(ex, axis=0, keepdims=True)
    s = jnp.sum(gates, axis=0, keepdims=True)

    y = jnp.dot(
        wexp_ref[...].astype(jnp.bfloat16),
        xb16,
        preferred_element_type=jnp.float32,
    )
    o_ref[0] = (y + bexp_ref[...]) * s


def kernel(x, We, be, Wn, bn, Wexp, bexp, noise_uniform):
    B, C, N, P = x.shape
    E = We.shape[0]
    O = Wexp.shape[0]
    NP = N * P
    T = 2048

    x3 = x.reshape(B, C, NP)
    u3 = noise_uniform.reshape(B, NP, E)
    bexp2 = bexp.reshape(O, 1)
    Wg = jnp.concatenate([We, Wn], axis=0)

    out = pl.pallas_call(
        _moe_block,
        grid=(B, NP // T),
        in_specs=[
            pl.BlockSpec((1, C, T), lambda b, i: (b, 0, i)),
            pl.BlockSpec((1, T, E), lambda b, i: (b, i, 0)),
            pl.BlockSpec((2 * E, C), lambda b, i: (0, 0)),
            pl.BlockSpec((O, C), lambda b, i: (0, 0)),
            pl.BlockSpec((O, 1), lambda b, i: (0, 0)),
        ],
        out_specs=pl.BlockSpec((1, O, T), lambda b, i: (b, 0, i)),
        out_shape=jax.ShapeDtypeStruct((B, O, NP), x.dtype),
    )(x3, u3, Wg, Wexp, bexp2)
    return out.reshape(B, O, N, P)


# speedup vs baseline: 1.2637x; 1.0266x over previous
import jax
import jax.numpy as jnp
from jax.experimental import pallas as pl


def _moe_block(x_ref, u_ref, wg_ref, wexp_ref, bexp_ref, o_ref):
    xb16 = x_ref[0].astype(jnp.bfloat16)
    E = u_ref.shape[1]
    ut = u_ref[0]

    g = jnp.dot(wg_ref[...].astype(jnp.bfloat16), xb16,
                preferred_element_type=jnp.float32)
    el = g[:E]
    nl = g[E:]
    nl_max = jnp.max(nl, axis=0, keepdims=True)
    nl_exp = jnp.exp(nl - nl_max)
    noise = ut * (nl_exp / jnp.sum(nl_exp, axis=0, keepdims=True))
    logits = el + noise

    lmin = jnp.min(logits, axis=0, keepdims=True)
    eidx = jax.lax.broadcasted_iota(jnp.int32, logits.shape, 0)
    drop = jnp.max(jnp.where(logits == lmin, eidx, -1), axis=0, keepdims=True)
    keep = eidx != drop
    lmax = jnp.max(logits, axis=0, keepdims=True)
    ex = jnp.where(keep, jnp.exp(logits - lmax), 0.0)
    gates = ex / jnp.sum(ex, axis=0, keepdims=True)
    s = jnp.sum(gates, axis=0, keepdims=True)

    y = jnp.dot(
        wexp_ref[...].astype(jnp.bfloat16),
        xb16,
        preferred_element_type=jnp.float32,
    )
    o_ref[0] = (y + bexp_ref[...]) * s


def kernel(x, We, be, Wn, bn, Wexp, bexp, noise_uniform):
    B, C, N, P = x.shape
    E = We.shape[0]
    O = Wexp.shape[0]
    NP = N * P
    T = 2048

    x3 = x.reshape(B, C, NP)
    u3 = jnp.transpose(noise_uniform.reshape(B, NP, E), (0, 2, 1))
    bexp2 = bexp.reshape(O, 1)
    Wg = jnp.concatenate([We, Wn], axis=0)

    out = pl.pallas_call(
        _moe_block,
        grid=(B, NP // T),
        in_specs=[
            pl.BlockSpec((1, C, T), lambda b, i: (b, 0, i)),
            pl.BlockSpec((1, E, T), lambda b, i: (b, 0, i)),
            pl.BlockSpec((2 * E, C), lambda b, i: (0, 0)),
            pl.BlockSpec((O, C), lambda b, i: (0, 0)),
            pl.BlockSpec((O, 1), lambda b, i: (0, 0)),
        ],
        out_specs=pl.BlockSpec((1, O, T), lambda b, i: (b, 0, i)),
        out_shape=jax.ShapeDtypeStruct((B, O, NP), x.dtype),
    )(x3, u3, Wg, Wexp, bexp2)
    return out.reshape(B, O, N, P)
